# baseline (speedup 1.0000x reference)
import functools

import jax
import jax.numpy as jnp
from jax import lax
from jax.experimental import pallas as pl
from jax.experimental.pallas import tpu as pltpu
from jax.experimental.pallas import tpu_sc as plsc

_B = 16384
_DM = 512
_DX = 1024
_NC = 2
_NS = 16
_NW = _NC * _NS
_P = 4
_BP = _B // _P
_RPW = _BP // _NW
_CHUNK = 64
_NCHUNK = _RPW // _CHUNK


def _sc_gather_body(nodes_hbm, mail_hbm, mem_hbm, lu_hbm,
                    x_out, h_out, lu_out,
                    idx_v, xbuf, hbuf, lubuf, semx, semh, seml):
    wid = lax.axis_index("s") * _NC + lax.axis_index("c")
    pltpu.sync_copy(nodes_hbm.at[wid], idx_v)
    for j in range(_NCHUNK):
        base = wid * _RPW + j * _CHUNK
        cx = pltpu.async_copy(mail_hbm.at[idx_v.at[j]], xbuf, semx)
        ch = pltpu.async_copy(mem_hbm.at[idx_v.at[j]], hbuf, semh)
        cl = pltpu.async_copy(lu_hbm.at[idx_v.at[j]], lubuf, seml)
        cx.wait()
        ch.wait()
        cl.wait()
        pltpu.sync_copy(xbuf, x_out.at[pl.ds(base, _CHUNK)])
        pltpu.sync_copy(hbuf, h_out.at[pl.ds(base, _CHUNK)])
        pltpu.sync_copy(lubuf, lu_out.at[pl.ds(base, _CHUNK)])


_sc_gather = functools.partial(
    pl.kernel,
    out_type=(
        jax.ShapeDtypeStruct((_BP, 8, 128), jnp.float32),
        jax.ShapeDtypeStruct((_BP, _DM), jnp.float32),
        jax.ShapeDtypeStruct((_BP,), jnp.float32),
    ),
    mesh=plsc.VectorSubcoreMesh(
        core_axis_name="c", subcore_axis_name="s",
        num_cores=_NC, num_subcores=_NS),
    scratch_types=[
        pltpu.VMEM((_NCHUNK, _CHUNK), jnp.int32),
        pltpu.VMEM((_CHUNK, 8, 128), jnp.float32),
        pltpu.VMEM((_CHUNK, _DM), jnp.float32),
        pltpu.VMEM((_CHUNK,), jnp.float32),
        pltpu.SemaphoreType.DMA,
        pltpu.SemaphoreType.DMA,
        pltpu.SemaphoreType.DMA,
    ],
)(_sc_gather_body)


def _gru_body(x_ref, h_ref, wi_ref, wh_ref, bi_ref, bh_ref, o_ref):
    h = h_ref[...]
    x = x_ref[...].reshape(x_ref.shape[0], _DX).astype(jnp.bfloat16)
    gi = lax.dot_general(x, wi_ref[...], (((1,), (1,)), ((), ())),
                         preferred_element_type=jnp.float32) + bi_ref[...]
    gh = lax.dot_general(h.astype(jnp.bfloat16), wh_ref[...],
                         (((1,), (1,)), ((), ())),
                         preferred_element_type=jnp.float32) + bh_ref[...]
    r = jax.nn.sigmoid(gi[:, :_DM] + gh[:, :_DM])
    z = jax.nn.sigmoid(gi[:, _DM:2 * _DM] + gh[:, _DM:2 * _DM])
    n = jnp.tanh(gi[:, 2 * _DM:] + r * gh[:, 2 * _DM:])
    o_ref[...] = (1.0 - z) * n + z * h


_BM = 2048
_SPP = _BP // _BM


def _tc_gru_part(p, acc, x3, h, wi, wh, bi2, bh2):
    data_specs = [
        pl.BlockSpec((_BM, 8, 128), lambda i: (i, 0, 0)),
        pl.BlockSpec((_BM, _DM), lambda i: (i, 0)),
        pl.BlockSpec((3 * _DM, _DX), lambda i: (0, 0)),
        pl.BlockSpec((3 * _DM, _DM), lambda i: (0, 0)),
        pl.BlockSpec((1, 3 * _DM), lambda i: (0, 0)),
        pl.BlockSpec((1, 3 * _DM), lambda i: (0, 0)),
    ]
    if p == 0:
        body, in_specs, args = _gru_body, data_specs, ()
        aliases = {}
    else:
        body = lambda acc_ref, *refs: _gru_body(*refs)
        in_specs = [pl.BlockSpec(memory_space=pl.ANY)] + data_specs
        args = (acc,)
        aliases = {0: 0}
    return pl.pallas_call(
        body,
        grid=(_SPP,),
        in_specs=in_specs,
        out_specs=pl.BlockSpec((_BM, _DM), lambda i, p=p: (p * _SPP + i, 0)),
        out_shape=jax.ShapeDtypeStruct((_B, _DM), jnp.float32),
        input_output_aliases=aliases,
        compiler_params=pltpu.CompilerParams(
            dimension_semantics=("arbitrary",)),
    )(*args, x3, h, wi, wh, bi2, bh2)


def kernel(nodes, memory, mailbox, last_update, W_ih, W_hh, b_ih, b_hh):
    n_nodes = memory.shape[0]
    mail3 = mailbox.reshape(n_nodes, 8, 128)
    nodes4 = nodes.reshape(_P, _NW, _NCHUNK, _CHUNK)
    wi = W_ih.astype(jnp.bfloat16)
    wh = W_hh.astype(jnp.bfloat16)
    bi2 = b_ih.reshape(1, 3 * _DM)
    bh2 = b_hh.reshape(1, 3 * _DM)
    parts = [_sc_gather(nodes4[p], mail3, memory, last_update)
             for p in range(_P)]
    acc = None
    for p in range(_P):
        x3, h, _ = parts[p]
        acc = _tc_gru_part(p, acc, x3, h, wi, wh, bi2, bh2)
    lu = jnp.concatenate([parts[p][2] for p in range(_P)])
    return acc, lu


# speedup vs baseline: 1.1285x; 1.1285x over previous
import functools

import jax
import jax.numpy as jnp
from jax import lax
from jax.experimental import pallas as pl
from jax.experimental.pallas import tpu as pltpu
from jax.experimental.pallas import tpu_sc as plsc

_B = 16384
_DM = 512
_DX = 1024
_NC = 2
_NS = 16
_NW = _NC * _NS
_P = 4
_BP = _B // _P
_RPW = _BP // _NW
_CHUNK = 64
_NCHUNK = _RPW // _CHUNK


def _sc_gather_body(nodes_hbm, mail_hbm, mem_hbm, lu_hbm,
                    x_out, h_out, lu_out,
                    idx_v, xbuf, hbuf, lubuf, semx, semh, seml):
    wid = lax.axis_index("s") * _NC + lax.axis_index("c")
    pltpu.sync_copy(nodes_hbm.at[wid], idx_v)
    for j in range(_NCHUNK):
        base = wid * _RPW + j * _CHUNK
        cx = pltpu.async_copy(mail_hbm.at[idx_v.at[j]], xbuf, semx)
        ch = pltpu.async_copy(mem_hbm.at[idx_v.at[j]], hbuf, semh)
        cl = pltpu.async_copy(lu_hbm.at[idx_v.at[j]], lubuf, seml)
        cx.wait()
        ch.wait()
        cl.wait()
        pltpu.sync_copy(xbuf, x_out.at[pl.ds(base, _CHUNK)])
        pltpu.sync_copy(hbuf, h_out.at[pl.ds(base, _CHUNK)])
        pltpu.sync_copy(lubuf, lu_out.at[pl.ds(base, _CHUNK)])


_sc_gather = functools.partial(
    pl.kernel,
    out_type=(
        jax.ShapeDtypeStruct((_BP, 8, 128), jnp.float32),
        jax.ShapeDtypeStruct((_BP, _DM), jnp.float32),
        jax.ShapeDtypeStruct((_BP,), jnp.float32),
    ),
    mesh=plsc.VectorSubcoreMesh(
        core_axis_name="c", subcore_axis_name="s",
        num_cores=_NC, num_subcores=_NS),
    scratch_types=[
        pltpu.VMEM((_NCHUNK, _CHUNK), jnp.int32),
        pltpu.VMEM((_CHUNK, 8, 128), jnp.float32),
        pltpu.VMEM((_CHUNK, _DM), jnp.float32),
        pltpu.VMEM((_CHUNK,), jnp.float32),
        pltpu.SemaphoreType.DMA,
        pltpu.SemaphoreType.DMA,
        pltpu.SemaphoreType.DMA,
    ],
)(_sc_gather_body)


def _gru_body(x_ref, h_ref, wi_ref, wh_ref, bi_ref, bh_ref, o_ref):
    h = h_ref[...]
    x = x_ref[...].reshape(x_ref.shape[0], _DX).astype(jnp.bfloat16)
    gi = lax.dot_general(x, wi_ref[...], (((1,), (1,)), ((), ())),
                         preferred_element_type=jnp.float32) + bi_ref[...]
    gh = lax.dot_general(h.astype(jnp.bfloat16), wh_ref[...],
                         (((1,), (1,)), ((), ())),
                         preferred_element_type=jnp.float32) + bh_ref[...]
    r = jax.nn.sigmoid(gi[:, :_DM] + gh[:, :_DM])
    z = jax.nn.sigmoid(gi[:, _DM:2 * _DM] + gh[:, _DM:2 * _DM])
    n = jnp.tanh(gi[:, 2 * _DM:] + r * gh[:, 2 * _DM:])
    o_ref[...] = (1.0 - z) * n + z * h


_BM = 1024
_SPP = _BP // _BM


def _tc_gru_part(p, acc, x3, h, wi, wh, bi2, bh2):
    data_specs = [
        pl.BlockSpec((_BM, 8, 128), lambda i: (i, 0, 0)),
        pl.BlockSpec((_BM, _DM), lambda i: (i, 0)),
        pl.BlockSpec((3 * _DM, _DX), lambda i: (0, 0)),
        pl.BlockSpec((3 * _DM, _DM), lambda i: (0, 0)),
        pl.BlockSpec((1, 3 * _DM), lambda i: (0, 0)),
        pl.BlockSpec((1, 3 * _DM), lambda i: (0, 0)),
    ]
    if p == 0:
        body, in_specs, args = _gru_body, data_specs, ()
        aliases = {}
    else:
        body = lambda acc_ref, *refs: _gru_body(*refs)
        in_specs = [pl.BlockSpec(memory_space=pl.ANY)] + data_specs
        args = (acc,)
        aliases = {0: 0}
    return pl.pallas_call(
        body,
        grid=(_SPP,),
        in_specs=in_specs,
        out_specs=pl.BlockSpec((_BM, _DM), lambda i, p=p: (p * _SPP + i, 0)),
        out_shape=jax.ShapeDtypeStruct((_B, _DM), jnp.float32),
        input_output_aliases=aliases,
        compiler_params=pltpu.CompilerParams(
            dimension_semantics=("arbitrary",)),
    )(*args, x3, h, wi, wh, bi2, bh2)


def kernel(nodes, memory, mailbox, last_update, W_ih, W_hh, b_ih, b_hh):
    n_nodes = memory.shape[0]
    mail3 = mailbox.reshape(n_nodes, 8, 128)
    nodes4 = nodes.reshape(_P, _NW, _NCHUNK, _CHUNK)
    wi = W_ih.astype(jnp.bfloat16)
    wh = W_hh.astype(jnp.bfloat16)
    bi2 = b_ih.reshape(1, 3 * _DM)
    bh2 = b_hh.reshape(1, 3 * _DM)
    parts = [_sc_gather(nodes4[p], mail3, memory, last_update)
             for p in range(_P)]
    acc = None
    for p in range(_P):
        x3, h, _ = parts[p]
        acc = _tc_gru_part(p, acc, x3, h, wi, wh, bi2, bh2)
    lu = jnp.concatenate([parts[p][2] for p in range(_P)])
    return acc, lu


# speedup vs baseline: 1.1300x; 1.0014x over previous
import functools

import jax
import jax.numpy as jnp
from jax import lax
from jax.experimental import pallas as pl
from jax.experimental.pallas import tpu as pltpu
from jax.experimental.pallas import tpu_sc as plsc

_B = 16384
_DM = 512
_DX = 1024
_NC = 2
_NS = 16
_NW = _NC * _NS
_P = 4
_BP = _B // _P
_RPW = _BP // _NW
_CHUNK = 64
_NCHUNK = _RPW // _CHUNK


def _sc_gather_body(nodes_hbm, mail_hbm, mem_hbm, lu_hbm,
                    x_out, h_out, lu_out,
                    idx_v, xbuf, hbuf, lubuf, semx, semh, seml):
    wid = lax.axis_index("s") * _NC + lax.axis_index("c")
    pltpu.sync_copy(nodes_hbm.at[wid], idx_v)
    for j in range(_NCHUNK):
        base = wid * _RPW + j * _CHUNK
        cx = pltpu.async_copy(mail_hbm.at[idx_v.at[j]], xbuf, semx)
        ch = pltpu.async_copy(mem_hbm.at[idx_v.at[j]], hbuf, semh)
        cl = pltpu.async_copy(lu_hbm.at[idx_v.at[j]], lubuf, seml)
        cx.wait()
        ch.wait()
        cl.wait()
        pltpu.sync_copy(xbuf, x_out.at[pl.ds(base, _CHUNK)])
        pltpu.sync_copy(hbuf, h_out.at[pl.ds(base, _CHUNK)])
        pltpu.sync_copy(lubuf, lu_out.at[pl.ds(base, _CHUNK)])


_sc_gather = functools.partial(
    pl.kernel,
    out_type=(
        jax.ShapeDtypeStruct((_BP, 8, 128), jnp.float32),
        jax.ShapeDtypeStruct((_BP, _DM), jnp.float32),
        jax.ShapeDtypeStruct((_BP,), jnp.float32),
    ),
    mesh=plsc.VectorSubcoreMesh(
        core_axis_name="c", subcore_axis_name="s",
        num_cores=_NC, num_subcores=_NS),
    scratch_types=[
        pltpu.VMEM((_NCHUNK, _CHUNK), jnp.int32),
        pltpu.VMEM((_CHUNK, 8, 128), jnp.float32),
        pltpu.VMEM((_CHUNK, _DM), jnp.float32),
        pltpu.VMEM((_CHUNK,), jnp.float32),
        pltpu.SemaphoreType.DMA,
        pltpu.SemaphoreType.DMA,
        pltpu.SemaphoreType.DMA,
    ],
)(_sc_gather_body)


def _gru_body(x_ref, h_ref, wi_ref, wh_ref, bi_ref, bh_ref, o_ref):
    h = h_ref[...]
    x = x_ref[...].reshape(x_ref.shape[0], _DX).astype(jnp.bfloat16)
    gi = lax.dot_general(x, wi_ref[...], (((1,), (1,)), ((), ())),
                         preferred_element_type=jnp.float32) + bi_ref[...]
    gh = lax.dot_general(h.astype(jnp.bfloat16), wh_ref[...],
                         (((1,), (1,)), ((), ())),
                         preferred_element_type=jnp.float32) + bh_ref[...]
    r = jax.nn.sigmoid(gi[:, :_DM] + gh[:, :_DM])
    z = jax.nn.sigmoid(gi[:, _DM:2 * _DM] + gh[:, _DM:2 * _DM])
    n = jnp.tanh(gi[:, 2 * _DM:] + r * gh[:, 2 * _DM:])
    o_ref[...] = (1.0 - z) * n + z * h


_BM = 1024
_SPP = _BP // _BM


def _tc_gru_part(p, acc, x3, h, wi, wh, bi2, bh2):
    data_specs = [
        pl.BlockSpec((_BM, 8, 128), lambda i: (i, 0, 0)),
        pl.BlockSpec((_BM, _DM), lambda i: (i, 0)),
        pl.BlockSpec((3 * _DM, _DX), lambda i: (0, 0)),
        pl.BlockSpec((3 * _DM, _DM), lambda i: (0, 0)),
        pl.BlockSpec((1, 3 * _DM), lambda i: (0, 0)),
        pl.BlockSpec((1, 3 * _DM), lambda i: (0, 0)),
    ]
    if p == 0:
        body, in_specs, args = _gru_body, data_specs, ()
        aliases = {}
    else:
        body = lambda acc_ref, *refs: _gru_body(*refs)
        in_specs = [pl.BlockSpec(memory_space=pl.ANY)] + data_specs
        args = (acc,)
        aliases = {0: 0}
    return pl.pallas_call(
        body,
        grid=(_SPP,),
        in_specs=in_specs,
        out_specs=pl.BlockSpec((_BM, _DM), lambda i, p=p: (p * _SPP + i, 0)),
        out_shape=jax.ShapeDtypeStruct((_B, _DM), jnp.float32),
        input_output_aliases=aliases,
        compiler_params=pltpu.CompilerParams(
            dimension_semantics=("parallel",)),
    )(*args, x3, h, wi, wh, bi2, bh2)


def kernel(nodes, memory, mailbox, last_update, W_ih, W_hh, b_ih, b_hh):
    n_nodes = memory.shape[0]
    mail3 = mailbox.reshape(n_nodes, 8, 128)
    nodes4 = nodes.reshape(_P, _NW, _NCHUNK, _CHUNK)
    wi = W_ih.astype(jnp.bfloat16)
    wh = W_hh.astype(jnp.bfloat16)
    bi2 = b_ih.reshape(1, 3 * _DM)
    bh2 = b_hh.reshape(1, 3 * _DM)
    parts = [_sc_gather(nodes4[p], mail3, memory, last_update)
             for p in range(_P)]
    acc = None
    for p in range(_P):
        x3, h, _ = parts[p]
        acc = _tc_gru_part(p, acc, x3, h, wi, wh, bi2, bh2)
    lu = jnp.concatenate([parts[p][2] for p in range(_P)])
    return acc, lu
